# baseline (speedup 1.0000x reference)
import functools

import jax
import jax.numpy as jnp
from jax import lax
from jax.experimental import pallas as pl
from jax.experimental.pallas import tpu as pltpu
from jax.experimental.pallas import tpu_sc as plsc

_LANES = 128
_NBUF = 2
_BR = 8


def _scan_buffer(buf_ref, k, v):
    lane = lax.broadcasted_iota(jnp.int32, (_BR, _LANES), 1)
    nfull = v // _LANES

    def chunk_at(base):
        return buf_ref[k, :, base : base + _LANES]

    m = chunk_at(0)
    g = lane
    for j in range(1, nfull):
        base = j * _LANES
        chunk = chunk_at(base)
        upd = chunk > m
        m = jnp.where(upd, chunk, m)
        g = jnp.where(upd, lane + base, g)
    if v % _LANES:
        base = v - _LANES
        chunk = chunk_at(base)
        upd = chunk > m
        m = jnp.where(upd, chunk, m)
        g = jnp.where(upd, lane + base, g)

    rowmax = jnp.max(m, axis=1, keepdims=True)
    cand = jnp.where(m == rowmax, g, jnp.int32(v))
    return jnp.min(cand, axis=1)


def _argmax_manual_body(x_hbm, out_ref, buf, sems):
    b, v = x_hbm.shape
    nblocks = b // _BR

    def start(k, blk):
        pltpu.make_async_copy(
            x_hbm.at[pl.ds(blk * _BR, _BR), :], buf.at[k], sems.at[k]
        ).start(priority=k % 2)

    def wait(k):
        pltpu.make_async_copy(
            x_hbm.at[pl.ds(0, _BR), :], buf.at[k], sems.at[k]
        ).wait()

    for k in range(_NBUF):
        start(k, k)

    def macro(ms, _):
        for k in range(_NBUF):
            blk = ms * _NBUF + k
            wait(k)
            idx = _scan_buffer(buf, k, v)
            out_ref[pl.ds(blk * _BR, _BR), :] = idx.reshape(_BR, 1)
            nxt = blk + _NBUF

            @pl.when(nxt < nblocks)
            def _():
                start(k, nxt)

        return 0

    lax.fori_loop(0, nblocks // _NBUF, macro, 0)


def _argmax_tc(inputs):
    b, v = inputs.shape
    return pl.pallas_call(
        _argmax_manual_body,
        in_specs=[pl.BlockSpec(memory_space=pl.ANY)],
        out_specs=pl.BlockSpec(memory_space=pltpu.MemorySpace.VMEM),
        out_shape=jax.ShapeDtypeStruct((b, 1), jnp.int32),
        scratch_shapes=[
            pltpu.VMEM((_NBUF, _BR, v), jnp.float32),
            pltpu.SemaphoreType.DMA((_NBUF,)),
        ],
        compiler_params=pltpu.CompilerParams(vmem_limit_bytes=100 * 1024 * 1024),
    )(inputs)


def _gather_sc(embeddings, idx):
    (b,) = idx.shape
    v, d = embeddings.shape
    info = plsc.get_sparse_core_info()
    nw = info.num_cores * info.num_subcores
    assert b % (8 * nw) == 0 and d % info.num_lanes == 0
    b_per_w = b // nw
    mesh = plsc.VectorSubcoreMesh(core_axis_name="c", subcore_axis_name="s")

    @functools.partial(
        pl.kernel,
        mesh=mesh,
        out_type=jax.ShapeDtypeStruct((b, d), jnp.float32),
        scratch_types=[
            pltpu.VMEM((b_per_w,), jnp.int32),
            pltpu.VMEM((b_per_w, d), jnp.float32),
            pltpu.SemaphoreType.DMA,
        ],
        compiler_params=pltpu.CompilerParams(use_tc_tiling_on_sc=False),
    )
    def gather_kernel(table_hbm, idx_hbm, out_hbm, idx_v, rows_v, sem):
        wid = lax.axis_index("s") * info.num_cores + lax.axis_index("c")
        base = wid * b_per_w
        pltpu.sync_copy(idx_hbm.at[pl.ds(base, b_per_w)], idx_v)
        pltpu.async_copy(table_hbm.at[idx_v], rows_v, sem).wait()
        pltpu.sync_copy(rows_v, out_hbm.at[pl.ds(base, b_per_w)])

    return gather_kernel(embeddings, idx)


def kernel(inputs, embeddings):
    idx = _argmax_tc(inputs).reshape(inputs.shape[0])
    return _gather_sc(embeddings, idx)
